# Initial kernel scaffold; baseline (speedup 1.0000x reference)
import jax
import jax.numpy as jnp
from jax import lax
from jax.experimental import pallas as pl
from jax.experimental.pallas import tpu as pltpu
from jax.experimental.pallas import tpu_sc as plsc

N = 100000
E = 1600000
G = 64

NPAD = 100352
NW = 32
E_W = E // NW
K = 128
CH = (E_W + K - 1) // K
LW = CH * K
RPT = NPAD // 16
NZ = RPT // K


def _sc_scatter_kernel(D, gather):
  mesh = plsc.VectorSubcoreMesh(core_axis_name="c", subcore_axis_name="s")

  scratch = [pltpu.VMEM((CH, K), jnp.int32)]
  if gather:
    scratch.append(pltpu.VMEM((CH, K), jnp.int32))
  scratch += [
      pltpu.VMEM((K, D), jnp.float32),
      pltpu.VMEM_SHARED((NPAD, D), jnp.float32),
  ]

  def body(*refs):
    if gather:
      (table, src3, dst3, zrs, out, dst_v, src_v, rows_v, zsh) = refs
    else:
      (const_rows, zrs, dst3, out, dst_v, rows_v, zsh) = refs

    c = lax.axis_index("c")
    s = lax.axis_index("s")
    wid = c * 16 + s
    row0 = s * RPT

    pltpu.sync_copy(zrs, rows_v)

    @pl.loop(0, NZ)
    def _(i):
      pltpu.sync_copy(rows_v, zsh.at[pl.ds(row0 + i * K, K)])

    plsc.subcore_barrier()

    pltpu.sync_copy(dst3.at[wid], dst_v)
    if gather:
      pltpu.sync_copy(src3.at[wid], src_v)
    else:
      pltpu.sync_copy(const_rows, rows_v)

    @pl.loop(0, CH)
    def _(j):
      if gather:
        pltpu.sync_copy(table.at[src_v.at[j]], rows_v)
      pltpu.sync_copy(rows_v, zsh.at[dst_v.at[j]], add=True)

    plsc.subcore_barrier()

    @pl.loop(0, NZ)
    def _(i):
      pltpu.sync_copy(zsh.at[pl.ds(row0 + i * K, K)],
                      out.at[c, pl.ds(row0 + i * K, K)])

  return pl.kernel(
      body,
      out_type=jax.ShapeDtypeStruct((2, NPAD, D), jnp.float32),
      mesh=mesh,
      scratch_types=scratch,
  )


BN = 3584


def _tc_prep(dega, degb, x_pad):
  def body(da, db, xr, dinv_o, y1_o):
    deg = da[...] + db[...] + 1.0
    dinv = lax.rsqrt(deg)
    dinv_o[...] = dinv
    y1_o[...] = xr[...] * dinv

  grid = NPAD // BN
  return pl.pallas_call(
      body,
      grid=(grid,),
      in_specs=[
          pl.BlockSpec((BN, 1), lambda i: (i, 0)),
          pl.BlockSpec((BN, 1), lambda i: (i, 0)),
          pl.BlockSpec((BN, 2), lambda i: (i, 0)),
      ],
      out_specs=[
          pl.BlockSpec((BN, 1), lambda i: (i, 0)),
          pl.BlockSpec((BN, 2), lambda i: (i, 0)),
      ],
      out_shape=[
          jax.ShapeDtypeStruct((NPAD, 1), jnp.float32),
          jax.ShapeDtypeStruct((NPAD, 2), jnp.float32),
      ],
  )(dega, degb, x_pad)


def _tc_layer1(z1a, z1b, y1, dinv, W1, b1):
  def body(za, zb, yr, dv, w1, b1r, y2_o):
    p1 = (za[...] + zb[...] + yr[...]) * dv[...]
    h = (p1[:, 0:1] * w1[0:1, :] + p1[:, 1:2] * w1[1:2, :] + b1r[...])
    y2_o[...] = jnp.maximum(h, 0.0) * dv[...]

  grid = NPAD // BN
  return pl.pallas_call(
      body,
      grid=(grid,),
      in_specs=[
          pl.BlockSpec((BN, 2), lambda i: (i, 0)),
          pl.BlockSpec((BN, 2), lambda i: (i, 0)),
          pl.BlockSpec((BN, 2), lambda i: (i, 0)),
          pl.BlockSpec((BN, 1), lambda i: (i, 0)),
          pl.BlockSpec((2, 16), lambda i: (0, 0)),
          pl.BlockSpec((1, 16), lambda i: (0, 0)),
      ],
      out_specs=pl.BlockSpec((BN, 16), lambda i: (i, 0)),
      out_shape=jax.ShapeDtypeStruct((NPAD, 16), jnp.float32),
  )(z1a, z1b, y1, dinv, W1, b1)


def _tc_head(z2a, z2b, y2, dinv, batch_pad, W2, b2, Wl, bl):
  grid = NPAD // BN

  def body(za, zb, yr, dv, bt, w2, b2r, wl, blr, out_o, sums, counts):
    i = pl.program_id(0)

    @pl.when(i == 0)
    def _():
      sums[...] = jnp.zeros_like(sums)
      counts[...] = jnp.zeros_like(counts)

    p2 = (za[...] + zb[...] + yr[...]) * dv[...]
    gids = lax.broadcasted_iota(jnp.int32, (BN, G), 1)
    onehot = jnp.where(bt[...] == gids, 1.0, 0.0)
    dn = (((0,), (0,)), ((), ()))
    sums[...] += lax.dot_general(onehot, p2, dn,
                                 preferred_element_type=jnp.float32)
    ones_col = jnp.ones((BN, 1), dtype=jnp.float32)
    counts[...] += lax.dot_general(onehot, ones_col, dn,
                                   preferred_element_type=jnp.float32)

    @pl.when(i == grid - 1)
    def _():
      pooled = sums[...] / jnp.maximum(counts[...], 1.0)
      w = jnp.dot(w2[...], wl[...],
                  preferred_element_type=jnp.float32)
      r = (jnp.dot(pooled, w, preferred_element_type=jnp.float32)
           + jnp.dot(b2r[...], wl[...], preferred_element_type=jnp.float32)
           + blr[...])
      out_o[...] = 1.0 / (1.0 + jnp.exp(-r))

  return pl.pallas_call(
      body,
      grid=(grid,),
      in_specs=[
          pl.BlockSpec((BN, 16), lambda i: (i, 0)),
          pl.BlockSpec((BN, 16), lambda i: (i, 0)),
          pl.BlockSpec((BN, 16), lambda i: (i, 0)),
          pl.BlockSpec((BN, 1), lambda i: (i, 0)),
          pl.BlockSpec((BN, 1), lambda i: (i, 0)),
          pl.BlockSpec((16, 32), lambda i: (0, 0)),
          pl.BlockSpec((1, 32), lambda i: (0, 0)),
          pl.BlockSpec((32, 1), lambda i: (0, 0)),
          pl.BlockSpec((1, 1), lambda i: (0, 0)),
      ],
      out_specs=pl.BlockSpec((G, 1), lambda i: (0, 0)),
      out_shape=jax.ShapeDtypeStruct((G, 1), jnp.float32),
      scratch_shapes=[
          pltpu.VMEM((G, 16), jnp.float32),
          pltpu.VMEM((G, 1), jnp.float32),
      ],
  )(z2a, z2b, y2, dinv, batch_pad, W2, b2, Wl, bl)


def kernel(x, edge_index, batch, W1, b1, W2, b2, Wl, bl):
  src = edge_index[0]
  dst = edge_index[1]

  pad = LW - E_W
  src3 = jnp.concatenate(
      [src.reshape(NW, E_W), jnp.zeros((NW, pad), jnp.int32)], axis=1
  ).reshape(NW, CH, K)
  dst3 = jnp.concatenate(
      [dst.reshape(NW, E_W), jnp.full((NW, pad), N, jnp.int32)], axis=1
  ).reshape(NW, CH, K)

  x_pad = jnp.pad(x, ((0, NPAD - N), (0, 0)))
  batch_pad = jnp.pad(batch, (0, NPAD - N),
                      constant_values=G).reshape(NPAD, 1)

  ones_1 = jnp.ones((K, 1), jnp.float32)
  zeros_1 = jnp.zeros((K, 1), jnp.float32)
  zeros_2 = jnp.zeros((K, 2), jnp.float32)
  zeros_16 = jnp.zeros((K, 16), jnp.float32)

  deg_p = _sc_scatter_kernel(1, gather=False)(ones_1, zeros_1, dst3)

  dinv, y1 = _tc_prep(deg_p[0], deg_p[1], x_pad)

  z1 = _sc_scatter_kernel(2, gather=True)(y1, src3, dst3, zeros_2)

  y2 = _tc_layer1(z1[0], z1[1], y1, dinv, W1, b1.reshape(1, 16))

  z2 = _sc_scatter_kernel(16, gather=True)(y2, src3, dst3, zeros_16)

  out = _tc_head(z2[0], z2[1], y2, dinv, batch_pad,
                 W2, b2.reshape(1, 32), Wl, bl.reshape(1, 1))
  return out


# speedup vs baseline: 8.7957x; 8.7957x over previous
import jax
import jax.numpy as jnp
from jax import lax
from jax.experimental import pallas as pl
from jax.experimental.pallas import tpu as pltpu
from jax.experimental.pallas import tpu_sc as plsc

N = 100000
E = 1600000
G = 64

NPAD = 100352
NW = 32
E_W = E // NW
K = 128
CH = (E_W + K - 1) // K
LW = CH * K
RPT = NPAD // 16
NZ = RPT // K

_MESH = dict(core_axis_name="c", subcore_axis_name="s")
_CP = pltpu.CompilerParams(use_tc_tiling_on_sc=False)


def _sc_deg_kernel():
  def body(ones_r, zrs, dst3, out, dst_v, rows_v, zsh):
    c = lax.axis_index("c")
    s = lax.axis_index("s")
    wid = c * 16 + s
    row0 = s * RPT

    pltpu.sync_copy(zrs, rows_v)

    @pl.loop(0, NZ)
    def _(i):
      pltpu.sync_copy(rows_v, zsh.at[pl.ds(row0 + i * K, K)])

    plsc.subcore_barrier()

    pltpu.sync_copy(dst3.at[wid], dst_v)
    pltpu.sync_copy(ones_r, rows_v)

    @pl.loop(0, CH)
    def _(j):
      pltpu.sync_copy(rows_v, zsh.at[dst_v.at[j]], add=True)

    plsc.subcore_barrier()

    @pl.loop(0, NZ)
    def _(i):
      pltpu.sync_copy(zsh.at[pl.ds(row0 + i * K, K)],
                      out.at[c, pl.ds(row0 + i * K, K)])

  return pl.kernel(
      body,
      out_type=jax.ShapeDtypeStruct((2, NPAD, 1), jnp.float32),
      mesh=plsc.VectorSubcoreMesh(**_MESH),
      scratch_types=[
          pltpu.VMEM((CH, K), jnp.int32),
          pltpu.VMEM((K, 1), jnp.float32),
          pltpu.VMEM_SHARED((NPAD, 1), jnp.float32),
      ],
      compiler_params=_CP,
  )


def _sc_prop_kernel(D, seg, srows, lo):
  srpt = srows // 16
  snz = srpt // K
  dum = srows - 16

  def body(tbl, src3, dst3, zrs, out,
           dst_v, sidx_a, sidx_b, dloc_a, dloc_b, rows_a, rows_b, zsh,
           sem_a, sem_b):
    c = lax.axis_index("c")
    s = lax.axis_index("s")
    wid = c * 16 + s
    row0 = s * srpt

    pltpu.sync_copy(zrs, rows_a)

    @pl.loop(0, snz)
    def _(i):
      pltpu.sync_copy(rows_a, zsh.at[pl.ds(row0 + i * K, K)])

    plsc.subcore_barrier()

    pltpu.sync_copy(dst3.at[wid], dst_v)

    def process(j, sidx, dloc, rows, sem):
      pltpu.sync_copy(src3.at[wid, j], sidx)
      pltpu.async_copy(tbl.at[sidx], rows, sem).wait()

      @pl.loop(0, K // 16)
      def _(g):
        d = dst_v[j, pl.ds(g * 16, 16)]
        local = d - lo
        ok = (local >= 0) & (local < seg)
        dloc[pl.ds(g * 16, 16)] = jnp.where(ok, local, dum)

      pltpu.sync_copy(rows, zsh.at[dloc], add=True)

    @pl.loop(0, CH // 2)
    def _(jj):
      process(jj * 2, sidx_a, dloc_a, rows_a, sem_a)
      process(jj * 2 + 1, sidx_b, dloc_b, rows_b, sem_b)

    process(CH - 1, sidx_a, dloc_a, rows_a, sem_a)

    plsc.subcore_barrier()

    @pl.loop(0, snz)
    def _(i):
      pltpu.sync_copy(zsh.at[pl.ds(row0 + i * K, K)],
                      out.at[c, pl.ds(row0 + i * K, K)])

  return pl.kernel(
      body,
      out_type=jax.ShapeDtypeStruct((2, srows, D), jnp.float32),
      mesh=plsc.VectorSubcoreMesh(**_MESH),
      scratch_types=[
          pltpu.VMEM((CH, K), jnp.int32),
          pltpu.VMEM((K,), jnp.int32),
          pltpu.VMEM((K,), jnp.int32),
          pltpu.VMEM((K,), jnp.int32),
          pltpu.VMEM((K,), jnp.int32),
          pltpu.VMEM((K, D), jnp.float32),
          pltpu.VMEM((K, D), jnp.float32),
          pltpu.VMEM_SHARED((srows, D), jnp.float32),
          pltpu.SemaphoreType.DMA,
          pltpu.SemaphoreType.DMA,
      ],
      compiler_params=_CP,
  )


HN = NPAD // 2
HROWS = 51200
QN = NPAD // 4
QROWS = 26624


BN = 3584


def _tc_prep(dega, degb, x_pad):
  def body(da, db, xr, dinv_o, y1_o):
    deg = da[...] + db[...] + 1.0
    dinv = lax.rsqrt(deg)
    dinv_o[...] = dinv
    y1 = xr[...] * dinv
    y1_o[...] = jnp.pad(y1, ((0, 0), (0, 6)))

  return pl.pallas_call(
      body,
      grid=(NPAD // BN,),
      in_specs=[
          pl.BlockSpec((BN, 1), lambda i: (i, 0)),
          pl.BlockSpec((BN, 1), lambda i: (i, 0)),
          pl.BlockSpec((BN, 2), lambda i: (i, 0)),
      ],
      out_specs=[
          pl.BlockSpec((BN, 1), lambda i: (i, 0)),
          pl.BlockSpec((BN, 8), lambda i: (i, 0)),
      ],
      out_shape=[
          jax.ShapeDtypeStruct((NPAD, 1), jnp.float32),
          jax.ShapeDtypeStruct((NPAD, 8), jnp.float32),
      ],
  )(dega, degb, x_pad)


def _tc_layer1(z1a, z1b, y1, dinv, W1, b1):
  def body(za, zb, yr, dv, w1, b1r, y2_o):
    p1 = (za[...] + zb[...] + yr[...]) * dv[...]
    h = (p1[:, 0:1] * w1[0:1, :] + p1[:, 1:2] * w1[1:2, :] + b1r[...])
    y2_o[...] = jnp.maximum(h, 0.0) * dv[...]

  return pl.pallas_call(
      body,
      grid=(NPAD // BN,),
      in_specs=[
          pl.BlockSpec((BN, 8), lambda i: (i, 0)),
          pl.BlockSpec((BN, 8), lambda i: (i, 0)),
          pl.BlockSpec((BN, 8), lambda i: (i, 0)),
          pl.BlockSpec((BN, 1), lambda i: (i, 0)),
          pl.BlockSpec((2, 16), lambda i: (0, 0)),
          pl.BlockSpec((1, 16), lambda i: (0, 0)),
      ],
      out_specs=pl.BlockSpec((BN, 16), lambda i: (i, 0)),
      out_shape=jax.ShapeDtypeStruct((NPAD, 16), jnp.float32),
  )(z1a, z1b, y1, dinv, W1, b1)


def _tc_head(z2a, z2b, y2, dinv, batch_pad, W2, b2, Wl, bl):
  grid = NPAD // BN

  def body(za, zb, yr, dv, bt, w2, b2r, wl, blr, out_o, sums, counts):
    i = pl.program_id(0)

    @pl.when(i == 0)
    def _():
      sums[...] = jnp.zeros_like(sums)
      counts[...] = jnp.zeros_like(counts)

    p2 = (za[...] + zb[...] + yr[...]) * dv[...]
    gids = lax.broadcasted_iota(jnp.int32, (BN, G), 1)
    onehot = jnp.where(bt[...] == gids, 1.0, 0.0)
    dn = (((0,), (0,)), ((), ()))
    sums[...] += lax.dot_general(onehot, p2, dn,
                                 preferred_element_type=jnp.float32)
    ones_col = jnp.ones((BN, 1), dtype=jnp.float32)
    counts[...] += lax.dot_general(onehot, ones_col, dn,
                                   preferred_element_type=jnp.float32)

    @pl.when(i == grid - 1)
    def _():
      pooled = sums[...] / jnp.maximum(counts[...], 1.0)
      w = jnp.dot(w2[...], wl[...],
                  preferred_element_type=jnp.float32)
      r = (jnp.dot(pooled, w, preferred_element_type=jnp.float32)
           + jnp.dot(b2r[...], wl[...], preferred_element_type=jnp.float32)
           + blr[...])
      out_o[...] = 1.0 / (1.0 + jnp.exp(-r))

  return pl.pallas_call(
      body,
      grid=(grid,),
      in_specs=[
          pl.BlockSpec((BN, 16), lambda i: (i, 0)),
          pl.BlockSpec((BN, 16), lambda i: (i, 0)),
          pl.BlockSpec((BN, 16), lambda i: (i, 0)),
          pl.BlockSpec((BN, 1), lambda i: (i, 0)),
          pl.BlockSpec((BN, 1), lambda i: (i, 0)),
          pl.BlockSpec((16, 32), lambda i: (0, 0)),
          pl.BlockSpec((1, 32), lambda i: (0, 0)),
          pl.BlockSpec((32, 1), lambda i: (0, 0)),
          pl.BlockSpec((1, 1), lambda i: (0, 0)),
      ],
      out_specs=pl.BlockSpec((G, 1), lambda i: (0, 0)),
      out_shape=jax.ShapeDtypeStruct((G, 1), jnp.float32),
      scratch_shapes=[
          pltpu.VMEM((G, 16), jnp.float32),
          pltpu.VMEM((G, 1), jnp.float32),
      ],
  )(z2a, z2b, y2, dinv, batch_pad, W2, b2, Wl, bl)


def kernel(x, edge_index, batch, W1, b1, W2, b2, Wl, bl):
  src = edge_index[0]
  dst = edge_index[1]

  pad = LW - E_W
  src3 = jnp.concatenate(
      [src.reshape(NW, E_W), jnp.zeros((NW, pad), jnp.int32)], axis=1
  ).reshape(NW, CH, K)
  dst3 = jnp.concatenate(
      [dst.reshape(NW, E_W), jnp.full((NW, pad), N, jnp.int32)], axis=1
  ).reshape(NW, CH, K)

  x_pad = jnp.pad(x, ((0, NPAD - N), (0, 0)))
  batch_pad = jnp.pad(batch, (0, NPAD - N),
                      constant_values=G).reshape(NPAD, 1)

  ones_1 = jnp.ones((K, 1), jnp.float32)
  zeros_1 = jnp.zeros((K, 1), jnp.float32)
  zeros_8 = jnp.zeros((K, 8), jnp.float32)
  zeros_16 = jnp.zeros((K, 16), jnp.float32)

  deg_p = _sc_deg_kernel()(ones_1, zeros_1, dst3)

  dinv, y1 = _tc_prep(deg_p[0], deg_p[1], x_pad)

  zh = [_sc_prop_kernel(8, HN, HROWS, h * HN)(y1, src3, dst3, zeros_8)
        for h in range(2)]
  z1a = jnp.concatenate([zh[h][0, :HN] for h in range(2)], axis=0)
  z1b = jnp.concatenate([zh[h][1, :HN] for h in range(2)], axis=0)

  y2 = _tc_layer1(z1a, z1b, y1, dinv, W1, b1.reshape(1, 16))

  zq = [_sc_prop_kernel(16, QN, QROWS, q * QN)(y2, src3, dst3, zeros_16)
        for q in range(4)]
  z2a = jnp.concatenate([zq[q][0, :QN] for q in range(4)], axis=0)
  z2b = jnp.concatenate([zq[q][1, :QN] for q in range(4)], axis=0)

  out = _tc_head(z2a, z2b, y2, dinv, batch_pad,
                 W2, b2.reshape(1, 32), Wl, bl.reshape(1, 1))
  return out
